# baseline (speedup 1.0000x reference)
import functools

import jax
import jax.numpy as jnp
from jax import lax
from jax.experimental import pallas as pl
from jax.experimental.pallas import tpu as pltpu
from jax.experimental.pallas import tpu_sc as plsc

B = 16384
V = 10
D = 512
NC = 2
NS = 16
NW = NC * NS
BPW = B // NW
CH = 64
NCHUNK = BPW // CH
L = 16

_mesh = plsc.VectorSubcoreMesh(
    core_axis_name="c", subcore_axis_name="s", num_cores=NC, num_subcores=NS
)


@functools.partial(
    pl.kernel,
    out_type=jax.ShapeDtypeStruct((B, D), jnp.int32),
    mesh=_mesh,
    scratch_types=[
        pltpu.VMEM((NCHUNK, CH), jnp.int32),
        pltpu.VMEM((2, CH, D), jnp.int32),
        pltpu.SemaphoreType.DMA,
        pltpu.SemaphoreType.DMA,
        pltpu.SemaphoreType.DMA,
        pltpu.SemaphoreType.DMA,
        pltpu.SemaphoreType.DMA,
        pltpu.SemaphoreType.DMA,
        pltpu.SemaphoreType.DMA,
        pltpu.SemaphoreType.DMA,
    ],
)
def _gather_sc(tbl_hbm, idx_hbm, out_hbm, idx_v, rows_v, g0, g1, g2, g3, s0, s1, s2, s3):
    wid = lax.axis_index("s") * NC + lax.axis_index("c")
    base = wid * BPW
    gsem = (g0, g1, g2, g3)
    ssem = (s0, s1, s2, s3)

    for c in range(NCHUNK):
        pltpu.sync_copy(idx_hbm.at[pl.ds(base + c * CH, CH)], idx_v.at[c])

    off = wid * V
    for c in range(NCHUNK):
        for j in range(CH // L):
            sl = pl.ds(j * L, L)
            idx_v[c, sl] = idx_v[c, sl] + off

    def fire_gather(c):
        return pltpu.async_copy(
            tbl_hbm.at[idx_v.at[c]], rows_v.at[c % 2], gsem[c % 2]
        )

    H = CH // 2

    def fire_store(c):
        slot = c % 2
        a = pltpu.async_copy(
            rows_v.at[slot].at[pl.ds(0, H)],
            out_hbm.at[pl.ds(base + c * CH, H)],
            ssem[2 * slot],
        )
        b = pltpu.async_copy(
            rows_v.at[slot].at[pl.ds(H, H)],
            out_hbm.at[pl.ds(base + c * CH + H, H)],
            ssem[2 * slot + 1],
        )
        return (a, b)

    gat = fire_gather(0)
    stores = {}
    for c in range(NCHUNK):
        if c + 1 < NCHUNK:
            if c - 1 >= 0:
                for h in stores[c - 1]:
                    h.wait()
            nxt = fire_gather(c + 1)
        gat.wait()
        stores[c] = fire_store(c)
        if c + 1 < NCHUNK:
            gat = nxt
    for c in (NCHUNK - 2, NCHUNK - 1):
        for h in stores[c]:
            h.wait()


def kernel(ind, mem):
    tbl = jnp.broadcast_to(mem.astype(jnp.int32), (NW, V, D)).reshape(NW * V, D)
    idx = ind.astype(jnp.int32)
    return _gather_sc(tbl, idx)


# speedup vs baseline: 4.0687x; 4.0687x over previous
import functools

import jax
import jax.numpy as jnp
from jax import lax
from jax.experimental import pallas as pl
from jax.experimental.pallas import tpu as pltpu

B = 16384
V = 10
D = 512
RB = 2048
NBLK = B // RB
VP = 16


def _tc_body(idx_ref, tbl_ref, out_ref):
    idxb = idx_ref[0]
    oh = (jnp.broadcast_to(idxb, (VP, RB))
          == lax.broadcasted_iota(jnp.int32, (VP, RB), 0)).astype(jnp.float32)
    vals = lax.dot_general(
        oh, tbl_ref[...],
        dimension_numbers=(((0,), (0,)), ((), ())),
        preferred_element_type=jnp.float32,
    )
    out_ref[...] = vals.astype(jnp.int32)


_tc_gather = pl.pallas_call(
    _tc_body,
    grid=(NBLK,),
    in_specs=[
        pl.BlockSpec((1, 1, RB), lambda i: (i, 0, 0)),
        pl.BlockSpec((VP, D), lambda i: (0, 0)),
    ],
    out_specs=pl.BlockSpec((RB, D), lambda i: (i, 0)),
    out_shape=jax.ShapeDtypeStruct((B, D), jnp.int32),
)


def kernel(ind, mem):
    idx3 = ind.astype(jnp.int32).reshape(NBLK, 1, RB)
    tblp = jnp.concatenate([mem, jnp.zeros((VP - V, D), jnp.float32)], axis=0)
    return _tc_gather(idx3, tblp)
